# Initial kernel scaffold; baseline (speedup 1.0000x reference)
import jax
import jax.numpy as jnp
from jax.experimental import pallas as pl


def kernel(xyz, points, W1, b1, W2, b2):
    raise NotImplementedError("write your pallas kernel here")


# speedup vs baseline: 20.8744x; 20.8744x over previous
import functools

import jax
import jax.numpy as jnp
from jax.experimental import pallas as pl
from jax.experimental.pallas import tpu as pltpu

K = 20
_INF = 3.0e38


def _body(xq_ref, xall_ref, pts_ref, W1_ref, W2_ref, b1_ref, b2_ref,
          out_ref, D_ref, Dm_ref, *, blk_q, n):
    q = pl.program_id(1)
    xq = xq_ref[0]
    xall = xall_ref[0]
    sq_all = jnp.sum(xall * xall, axis=0, keepdims=True)
    sq_q = jnp.sum(xq * xq, axis=1, keepdims=True)
    cross = jax.lax.dot_general(
        xq, xall, (((1,), (0,)), ((), ())),
        preferred_element_type=jnp.float32)
    D = sq_q + sq_all - 2.0 * cross
    D_ref[...] = D
    Dm_ref[...] = D

    def step(i, m):
        Dm = Dm_ref[...]
        m = jnp.min(Dm, axis=1, keepdims=True)
        Dm_ref[...] = jnp.where(Dm <= m, _INF, Dm)
        return m

    t = jax.lax.fori_loop(0, K + 1, step, jnp.zeros((blk_q, 1), jnp.float32))

    A = (D_ref[...] <= t).astype(jnp.float32)
    f_all = pts_ref[0]
    f_all = jnp.where(f_all > 0, f_all, 0.01 * f_all)
    agg = jax.lax.dot_general(
        A, f_all, (((1,), (0,)), ((), ())),
        preferred_element_type=jnp.float32)
    raw_q = pts_ref[0, pl.ds(q * blk_q, blk_q), :]
    f_q = jnp.where(raw_q > 0, raw_q, 0.01 * raw_q)
    agg_n = agg - f_q
    p1 = jax.lax.dot_general(
        f_q, W1_ref[...], (((1,), (1,)), ((), ())),
        preferred_element_type=jnp.float32)
    p2 = jax.lax.dot_general(
        agg_n, W2_ref[...], (((1,), (1,)), ((), ())),
        preferred_element_type=jnp.float32)
    bias = b1_ref[...] + jnp.float32(K) * b2_ref[...]
    out_ref[0] = (p1 + p2 + bias) * jnp.float32(1.0 / (K + 1)) + raw_q


def kernel(xyz, points, W1, b1, W2, b2):
    B, C, N = points.shape
    blk_q = 512
    xyzt = jnp.transpose(xyz, (0, 2, 1))
    pts_t = jnp.transpose(points, (0, 2, 1))
    grid = (B, N // blk_q)
    out_t = pl.pallas_call(
        functools.partial(_body, blk_q=blk_q, n=N),
        grid=grid,
        in_specs=[
            pl.BlockSpec((1, blk_q, 3), lambda b, q: (b, q, 0)),
            pl.BlockSpec((1, 3, N), lambda b, q: (b, 0, 0)),
            pl.BlockSpec((1, N, C), lambda b, q: (b, 0, 0)),
            pl.BlockSpec((C, C), lambda b, q: (0, 0)),
            pl.BlockSpec((C, C), lambda b, q: (0, 0)),
            pl.BlockSpec((1, C), lambda b, q: (0, 0)),
            pl.BlockSpec((1, C), lambda b, q: (0, 0)),
        ],
        out_specs=pl.BlockSpec((1, blk_q, C), lambda b, q: (b, q, 0)),
        out_shape=jax.ShapeDtypeStruct((B, N, C), jnp.float32),
        scratch_shapes=[
            pltpu.VMEM((blk_q, N), jnp.float32),
            pltpu.VMEM((blk_q, N), jnp.float32),
        ],
    )(xyzt, xyz, pts_t, W1, W2, b1.reshape(1, C), b2.reshape(1, C))
    return jnp.transpose(out_t, (0, 2, 1))


# speedup vs baseline: 24.2880x; 1.1635x over previous
import functools

import jax
import jax.numpy as jnp
from jax.experimental import pallas as pl
from jax.experimental.pallas import tpu as pltpu

K = 20
_INF = 3.0e38


def _body(xq_ref, xall_ref, pts_ref, W1_ref, W2_ref, b1_ref, b2_ref,
          out_ref, D_ref, *, blk_q, n):
    q = pl.program_id(1)
    xq = xq_ref[0]
    xall = xall_ref[0]
    sq_all = jnp.sum(xall * xall, axis=0, keepdims=True)
    sq_q = jnp.sum(xq * xq, axis=1, keepdims=True)
    cross = jax.lax.dot_general(
        xq, xall, (((1,), (0,)), ((), ())),
        preferred_element_type=jnp.float32)
    D = sq_q + sq_all - 2.0 * cross
    D_ref[...] = D

    def step(i, m):
        Dm = D_ref[...]
        return jnp.min(jnp.where(Dm > m, Dm, _INF), axis=1, keepdims=True)

    t = jax.lax.fori_loop(
        0, K + 1, step, jnp.full((blk_q, 1), -_INF, jnp.float32))

    A = (D_ref[...] <= t).astype(jnp.float32)
    f_all = pts_ref[0]
    f_all = jnp.where(f_all > 0, f_all, 0.01 * f_all)
    agg = jax.lax.dot_general(
        A, f_all, (((1,), (0,)), ((), ())),
        preferred_element_type=jnp.float32)
    raw_q = pts_ref[0, pl.ds(q * blk_q, blk_q), :]
    f_q = jnp.where(raw_q > 0, raw_q, 0.01 * raw_q)
    agg_n = agg - f_q
    p1 = jax.lax.dot_general(
        f_q, W1_ref[...], (((1,), (1,)), ((), ())),
        preferred_element_type=jnp.float32)
    p2 = jax.lax.dot_general(
        agg_n, W2_ref[...], (((1,), (1,)), ((), ())),
        preferred_element_type=jnp.float32)
    bias = b1_ref[...] + jnp.float32(K) * b2_ref[...]
    out_ref[0] = (p1 + p2 + bias) * jnp.float32(1.0 / (K + 1)) + raw_q


def kernel(xyz, points, W1, b1, W2, b2):
    B, C, N = points.shape
    blk_q = 512
    xyzt = jnp.transpose(xyz, (0, 2, 1))
    pts_t = jnp.transpose(points, (0, 2, 1))
    grid = (B, N // blk_q)
    out_t = pl.pallas_call(
        functools.partial(_body, blk_q=blk_q, n=N),
        grid=grid,
        in_specs=[
            pl.BlockSpec((1, blk_q, 3), lambda b, q: (b, q, 0)),
            pl.BlockSpec((1, 3, N), lambda b, q: (b, 0, 0)),
            pl.BlockSpec((1, N, C), lambda b, q: (b, 0, 0)),
            pl.BlockSpec((C, C), lambda b, q: (0, 0)),
            pl.BlockSpec((C, C), lambda b, q: (0, 0)),
            pl.BlockSpec((1, C), lambda b, q: (0, 0)),
            pl.BlockSpec((1, C), lambda b, q: (0, 0)),
        ],
        out_specs=pl.BlockSpec((1, blk_q, C), lambda b, q: (b, q, 0)),
        out_shape=jax.ShapeDtypeStruct((B, N, C), jnp.float32),
        scratch_shapes=[
            pltpu.VMEM((blk_q, N), jnp.float32),
        ],
    )(xyzt, xyz, pts_t, W1, W2, b1.reshape(1, C), b2.reshape(1, C))
    return jnp.transpose(out_t, (0, 2, 1))


# speedup vs baseline: 39.5063x; 1.6266x over previous
import functools

import jax
import jax.numpy as jnp
from jax.experimental import pallas as pl
from jax.experimental.pallas import tpu as pltpu

K = 20
_INF = 3.0e38


def _body(xq_ref, xall_ref, pts_ref, W1_ref, W2_ref, b1_ref, b2_ref,
          out_ref, D_ref, P_ref, *, blk_q, n):
    q = pl.program_id(1)
    xq = xq_ref[0]
    xall = xall_ref[0]
    sq_all = jnp.sum(xall * xall, axis=0, keepdims=True)
    sq_q = jnp.sum(xq * xq, axis=1, keepdims=True)
    cross = jax.lax.dot_general(
        xq, xall, (((1,), (0,)), ((), ())),
        preferred_element_type=jnp.float32)
    D = sq_q + sq_all - 2.0 * cross
    D_ref[...] = D

    nlev = 6
    nch = n // 128
    sub = 128

    def pyr_body(qq, _):
        rows = pl.ds(qq * sub, sub)
        prev = jnp.full((sub, 128), -_INF, jnp.float32)
        for j in range(nlev):
            acc = jnp.full((sub, 128), _INF, jnp.float32)
            for c in range(nch):
                x = D_ref[rows, pl.ds(c * 128, 128)]
                acc = jnp.minimum(acc, jnp.where(x > prev, x, _INF))
            P_ref[rows, pl.ds(j * 128, 128)] = acc
            prev = acc
        return 0

    jax.lax.fori_loop(0, blk_q // sub, pyr_body, 0, unroll=True)

    def step_small(i, m):
        P = P_ref[...]
        return jnp.min(jnp.where(P > m, P, _INF), axis=1, keepdims=True)

    t = jax.lax.fori_loop(
        0, K + 1, step_small, jnp.full((blk_q, 1), -_INF, jnp.float32))

    deepest = P_ref[:, pl.ds((nlev - 1) * 128, 128)]
    overflow = jnp.max(jnp.where(deepest < t, 1.0, 0.0))

    def slow_t():
        def step_full(i, m):
            Dm = D_ref[...]
            return jnp.min(jnp.where(Dm > m, Dm, _INF), axis=1, keepdims=True)
        return jax.lax.fori_loop(
            0, K + 1, step_full, jnp.full((blk_q, 1), -_INF, jnp.float32))

    t = jax.lax.cond(overflow > 0, slow_t, lambda: t)

    A = (D_ref[...] <= t).astype(jnp.float32)
    f_all = pts_ref[0]
    f_all = jnp.where(f_all > 0, f_all, 0.01 * f_all)
    agg = jax.lax.dot_general(
        A, f_all, (((1,), (0,)), ((), ())),
        preferred_element_type=jnp.float32)
    raw_q = pts_ref[0, pl.ds(q * blk_q, blk_q), :]
    f_q = jnp.where(raw_q > 0, raw_q, 0.01 * raw_q)
    agg_n = agg - f_q
    p1 = jax.lax.dot_general(
        f_q, W1_ref[...], (((1,), (1,)), ((), ())),
        preferred_element_type=jnp.float32)
    p2 = jax.lax.dot_general(
        agg_n, W2_ref[...], (((1,), (1,)), ((), ())),
        preferred_element_type=jnp.float32)
    bias = b1_ref[...] + jnp.float32(K) * b2_ref[...]
    out_ref[0] = (p1 + p2 + bias) * jnp.float32(1.0 / (K + 1)) + raw_q


def kernel(xyz, points, W1, b1, W2, b2):
    B, C, N = points.shape
    blk_q = 512
    xyzt = jnp.transpose(xyz, (0, 2, 1))
    pts_t = jnp.transpose(points, (0, 2, 1))
    grid = (B, N // blk_q)
    out_t = pl.pallas_call(
        functools.partial(_body, blk_q=blk_q, n=N),
        grid=grid,
        in_specs=[
            pl.BlockSpec((1, blk_q, 3), lambda b, q: (b, q, 0)),
            pl.BlockSpec((1, 3, N), lambda b, q: (b, 0, 0)),
            pl.BlockSpec((1, N, C), lambda b, q: (b, 0, 0)),
            pl.BlockSpec((C, C), lambda b, q: (0, 0)),
            pl.BlockSpec((C, C), lambda b, q: (0, 0)),
            pl.BlockSpec((1, C), lambda b, q: (0, 0)),
            pl.BlockSpec((1, C), lambda b, q: (0, 0)),
        ],
        out_specs=pl.BlockSpec((1, blk_q, C), lambda b, q: (b, q, 0)),
        out_shape=jax.ShapeDtypeStruct((B, N, C), jnp.float32),
        scratch_shapes=[
            pltpu.VMEM((blk_q, N), jnp.float32),
            pltpu.VMEM((blk_q, 6 * 128), jnp.float32),
        ],
    )(xyzt, xyz, pts_t, W1, W2, b1.reshape(1, C), b2.reshape(1, C))
    return jnp.transpose(out_t, (0, 2, 1))


# speedup vs baseline: 46.5797x; 1.1790x over previous
import functools

import jax
import jax.numpy as jnp
from jax.experimental import pallas as pl
from jax.experimental.pallas import tpu as pltpu

K = 20
_INF = 3.0e38


def _body(xq_ref, xall_ref, pts_ref, W1_ref, W2_ref, b1_ref, b2_ref,
          out_ref, D_ref, P_ref, *, blk_q, n):
    q = pl.program_id(1)
    xq = xq_ref[0]
    xall = xall_ref[0]
    sq_all = jnp.sum(xall * xall, axis=0, keepdims=True)
    sq_q = jnp.sum(xq * xq, axis=1, keepdims=True)
    cross = jax.lax.dot_general(
        xq, xall, (((1,), (0,)), ((), ())),
        preferred_element_type=jnp.float32)
    D = sq_q + sq_all - 2.0 * cross
    D_ref[...] = D

    nlev = 6
    nch = n // 128
    sub = 64

    def pyr_body(qq, _):
        rows = pl.ds(qq * sub, sub)
        lev = [jnp.full((sub, 128), _INF, jnp.float32) for _ in range(nlev)]
        for c in range(nch):
            y = D_ref[rows, pl.ds(c * 128, 128)]
            for j in range(nlev):
                lo = jnp.minimum(lev[j], y)
                y = jnp.maximum(lev[j], y)
                lev[j] = lo
        for j in range(nlev):
            P_ref[rows, pl.ds(j * 128, 128)] = lev[j]
        return 0

    jax.lax.fori_loop(0, blk_q // sub, pyr_body, 0, unroll=True)

    def step_small(i, m):
        P = P_ref[...]
        return jnp.min(jnp.where(P > m, P, _INF), axis=1, keepdims=True)

    t = jax.lax.fori_loop(
        0, K + 1, step_small, jnp.full((blk_q, 1), -_INF, jnp.float32))

    deepest = P_ref[:, pl.ds((nlev - 1) * 128, 128)]
    overflow = jnp.max(jnp.where(deepest < t, 1.0, 0.0))

    def slow_t():
        def step_full(i, m):
            Dm = D_ref[...]
            return jnp.min(jnp.where(Dm > m, Dm, _INF), axis=1, keepdims=True)
        return jax.lax.fori_loop(
            0, K + 1, step_full, jnp.full((blk_q, 1), -_INF, jnp.float32))

    t = jax.lax.cond(overflow > 0, slow_t, lambda: t)

    A = (D_ref[...] <= t).astype(jnp.float32)
    f_all = pts_ref[0]
    f_all = jnp.where(f_all > 0, f_all, 0.01 * f_all)
    agg = jax.lax.dot_general(
        A, f_all, (((1,), (0,)), ((), ())),
        preferred_element_type=jnp.float32)
    raw_q = pts_ref[0, pl.ds(q * blk_q, blk_q), :]
    f_q = jnp.where(raw_q > 0, raw_q, 0.01 * raw_q)
    agg_n = agg - f_q
    p1 = jax.lax.dot_general(
        f_q, W1_ref[...], (((1,), (1,)), ((), ())),
        preferred_element_type=jnp.float32)
    p2 = jax.lax.dot_general(
        agg_n, W2_ref[...], (((1,), (1,)), ((), ())),
        preferred_element_type=jnp.float32)
    bias = b1_ref[...] + jnp.float32(K) * b2_ref[...]
    out_ref[0] = (p1 + p2 + bias) * jnp.float32(1.0 / (K + 1)) + raw_q


def kernel(xyz, points, W1, b1, W2, b2):
    B, C, N = points.shape
    blk_q = 512
    xyzt = jnp.transpose(xyz, (0, 2, 1))
    pts_t = jnp.transpose(points, (0, 2, 1))
    grid = (B, N // blk_q)
    out_t = pl.pallas_call(
        functools.partial(_body, blk_q=blk_q, n=N),
        grid=grid,
        in_specs=[
            pl.BlockSpec((1, blk_q, 3), lambda b, q: (b, q, 0)),
            pl.BlockSpec((1, 3, N), lambda b, q: (b, 0, 0)),
            pl.BlockSpec((1, N, C), lambda b, q: (b, 0, 0)),
            pl.BlockSpec((C, C), lambda b, q: (0, 0)),
            pl.BlockSpec((C, C), lambda b, q: (0, 0)),
            pl.BlockSpec((1, C), lambda b, q: (0, 0)),
            pl.BlockSpec((1, C), lambda b, q: (0, 0)),
        ],
        out_specs=pl.BlockSpec((1, blk_q, C), lambda b, q: (b, q, 0)),
        out_shape=jax.ShapeDtypeStruct((B, N, C), jnp.float32),
        scratch_shapes=[
            pltpu.VMEM((blk_q, N), jnp.float32),
            pltpu.VMEM((blk_q, 6 * 128), jnp.float32),
        ],
    )(xyzt, xyz, pts_t, W1, W2, b1.reshape(1, C), b2.reshape(1, C))
    return jnp.transpose(out_t, (0, 2, 1))


# speedup vs baseline: 50.9745x; 1.0943x over previous
import functools

import jax
import jax.numpy as jnp
from jax.experimental import pallas as pl
from jax.experimental.pallas import tpu as pltpu

K = 20
_INF = 3.0e38


def _body(xq_ref, xall_ref, pts_ref, W1_ref, W2_ref, b1_ref, b2_ref,
          out_ref, D_ref, P_ref, *, blk_q, n):
    q = pl.program_id(1)
    xq = xq_ref[0]
    xall = xall_ref[0]
    sq_all = jnp.sum(xall * xall, axis=0, keepdims=True)
    sq_q = jnp.sum(xq * xq, axis=1, keepdims=True)
    cross = jax.lax.dot_general(
        xq, xall, (((1,), (0,)), ((), ())),
        preferred_element_type=jnp.float32)
    D = sq_q + sq_all - 2.0 * cross
    D_ref[...] = D

    nlev = 5
    nch = n // 128
    sub = 64

    def pyr_body(qq, _):
        rows = pl.ds(qq * sub, sub)
        lev = [jnp.full((sub, 128), _INF, jnp.float32) for _ in range(nlev)]
        for c in range(nch):
            y = D_ref[rows, pl.ds(c * 128, 128)]
            for j in range(nlev):
                lo = jnp.minimum(lev[j], y)
                y = jnp.maximum(lev[j], y)
                lev[j] = lo
        for j in range(nlev):
            P_ref[rows, pl.ds(j * 128, 128)] = lev[j]
        return 0

    jax.lax.fori_loop(0, blk_q // sub, pyr_body, 0, unroll=True)

    def step_small(i, m):
        P = P_ref[...]
        return jnp.min(jnp.where(P > m, P, _INF), axis=1, keepdims=True)

    t = jax.lax.fori_loop(
        0, K + 1, step_small, jnp.full((blk_q, 1), -_INF, jnp.float32))

    deepest = P_ref[:, pl.ds((nlev - 1) * 128, 128)]
    overflow = jnp.max(jnp.where(deepest < t, 1.0, 0.0))

    def slow_t():
        def step_full(i, m):
            Dm = D_ref[...]
            return jnp.min(jnp.where(Dm > m, Dm, _INF), axis=1, keepdims=True)
        return jax.lax.fori_loop(
            0, K + 1, step_full, jnp.full((blk_q, 1), -_INF, jnp.float32))

    t = jax.lax.cond(overflow > 0, slow_t, lambda: t)

    A = (D_ref[...] <= t).astype(jnp.float32)
    f_all = pts_ref[0]
    f_all = jnp.where(f_all > 0, f_all, 0.01 * f_all)
    agg = jax.lax.dot_general(
        A, f_all, (((1,), (0,)), ((), ())),
        preferred_element_type=jnp.float32)
    raw_q = pts_ref[0, pl.ds(q * blk_q, blk_q), :]
    f_q = jnp.where(raw_q > 0, raw_q, 0.01 * raw_q)
    agg_n = agg - f_q
    p1 = jax.lax.dot_general(
        f_q, W1_ref[...], (((1,), (1,)), ((), ())),
        preferred_element_type=jnp.float32)
    p2 = jax.lax.dot_general(
        agg_n, W2_ref[...], (((1,), (1,)), ((), ())),
        preferred_element_type=jnp.float32)
    bias = b1_ref[...] + jnp.float32(K) * b2_ref[...]
    out_ref[0] = (p1 + p2 + bias) * jnp.float32(1.0 / (K + 1)) + raw_q


def kernel(xyz, points, W1, b1, W2, b2):
    B, C, N = points.shape
    blk_q = 512
    xyzt = jnp.transpose(xyz, (0, 2, 1))
    pts_t = jnp.transpose(points, (0, 2, 1))
    grid = (B, N // blk_q)
    out_t = pl.pallas_call(
        functools.partial(_body, blk_q=blk_q, n=N),
        grid=grid,
        in_specs=[
            pl.BlockSpec((1, blk_q, 3), lambda b, q: (b, q, 0)),
            pl.BlockSpec((1, 3, N), lambda b, q: (b, 0, 0)),
            pl.BlockSpec((1, N, C), lambda b, q: (b, 0, 0)),
            pl.BlockSpec((C, C), lambda b, q: (0, 0)),
            pl.BlockSpec((C, C), lambda b, q: (0, 0)),
            pl.BlockSpec((1, C), lambda b, q: (0, 0)),
            pl.BlockSpec((1, C), lambda b, q: (0, 0)),
        ],
        out_specs=pl.BlockSpec((1, blk_q, C), lambda b, q: (b, q, 0)),
        out_shape=jax.ShapeDtypeStruct((B, N, C), jnp.float32),
        scratch_shapes=[
            pltpu.VMEM((blk_q, N), jnp.float32),
            pltpu.VMEM((blk_q, 5 * 128), jnp.float32),
        ],
    )(xyzt, xyz, pts_t, W1, W2, b1.reshape(1, C), b2.reshape(1, C))
    return jnp.transpose(out_t, (0, 2, 1))


# speedup vs baseline: 56.1987x; 1.1025x over previous
import functools

import jax
import jax.numpy as jnp
from jax.experimental import pallas as pl
from jax.experimental.pallas import tpu as pltpu

K = 20
_INF = 3.0e38


def _body(xq_ref, xall_ref, pts_ref, W1_ref, W2_ref, b1_ref, b2_ref,
          out_ref, D_ref, P_ref, *, blk_q, n):
    q = pl.program_id(1)
    xq = xq_ref[0]
    xall = xall_ref[0]
    sq_all = jnp.sum(xall * xall, axis=0, keepdims=True)
    sq_q = jnp.sum(xq * xq, axis=1, keepdims=True)
    cross = jax.lax.dot_general(
        xq, xall, (((1,), (0,)), ((), ())),
        preferred_element_type=jnp.float32)
    D = sq_q + sq_all - 2.0 * cross
    D_ref[...] = D

    nlev = 5
    nch = n // 128
    sub = 64

    def pyr_body(qq, _):
        rows = pl.ds(qq * sub, sub)
        lev = [jnp.full((sub, 128), _INF, jnp.float32) for _ in range(nlev)]
        for c in range(nch):
            y = D_ref[rows, pl.ds(c * 128, 128)]
            for j in range(nlev):
                lo = jnp.minimum(lev[j], y)
                y = jnp.maximum(lev[j], y)
                lev[j] = lo
        for j in range(nlev):
            P_ref[rows, pl.ds(j * 128, 128)] = lev[j]
        return 0

    jax.lax.fori_loop(0, blk_q // sub, pyr_body, 0, unroll=True)

    def step_small(i, m):
        P = P_ref[...]
        return jnp.min(jnp.where(P > m, P, _INF), axis=1, keepdims=True)

    t = jax.lax.fori_loop(
        0, K + 1, step_small, jnp.full((blk_q, 1), -_INF, jnp.float32))

    deepest = P_ref[:, pl.ds((nlev - 1) * 128, 128)]
    overflow = jnp.max(jnp.where(deepest < t, 1.0, 0.0))

    def slow_t():
        def step_full(i, m):
            Dm = D_ref[...]
            return jnp.min(jnp.where(Dm > m, Dm, _INF), axis=1, keepdims=True)
        return jax.lax.fori_loop(
            0, K + 1, step_full, jnp.full((blk_q, 1), -_INF, jnp.float32))

    t = jax.lax.cond(overflow > 0, slow_t, lambda: t)

    A = (D_ref[...] <= t).astype(jnp.float32)
    f_all = pts_ref[0]
    f_all = jnp.where(f_all > 0, f_all, 0.01 * f_all)
    agg = jax.lax.dot_general(
        A, f_all, (((1,), (0,)), ((), ())),
        preferred_element_type=jnp.float32)
    raw_q = pts_ref[0, pl.ds(q * blk_q, blk_q), :]
    f_q = jnp.where(raw_q > 0, raw_q, 0.01 * raw_q)
    agg_n = agg - f_q
    p1 = jax.lax.dot_general(
        f_q, W1_ref[...], (((1,), (1,)), ((), ())),
        preferred_element_type=jnp.float32)
    p2 = jax.lax.dot_general(
        agg_n, W2_ref[...], (((1,), (1,)), ((), ())),
        preferred_element_type=jnp.float32)
    bias = b1_ref[...] + jnp.float32(K) * b2_ref[...]
    out_ref[0] = (p1 + p2 + bias) * jnp.float32(1.0 / (K + 1)) + raw_q


def kernel(xyz, points, W1, b1, W2, b2):
    B, C, N = points.shape
    blk_q = 1024
    xyzt = jnp.transpose(xyz, (0, 2, 1))
    pts_t = jnp.transpose(points, (0, 2, 1))
    grid = (B, N // blk_q)
    out_t = pl.pallas_call(
        functools.partial(_body, blk_q=blk_q, n=N),
        grid=grid,
        in_specs=[
            pl.BlockSpec((1, blk_q, 3), lambda b, q: (b, q, 0)),
            pl.BlockSpec((1, 3, N), lambda b, q: (b, 0, 0)),
            pl.BlockSpec((1, N, C), lambda b, q: (b, 0, 0)),
            pl.BlockSpec((C, C), lambda b, q: (0, 0)),
            pl.BlockSpec((C, C), lambda b, q: (0, 0)),
            pl.BlockSpec((1, C), lambda b, q: (0, 0)),
            pl.BlockSpec((1, C), lambda b, q: (0, 0)),
        ],
        out_specs=pl.BlockSpec((1, blk_q, C), lambda b, q: (b, q, 0)),
        out_shape=jax.ShapeDtypeStruct((B, N, C), jnp.float32),
        scratch_shapes=[
            pltpu.VMEM((blk_q, N), jnp.float32),
            pltpu.VMEM((blk_q, 5 * 128), jnp.float32),
        ],
    )(xyzt, xyz, pts_t, W1, W2, b1.reshape(1, C), b2.reshape(1, C))
    return jnp.transpose(out_t, (0, 2, 1))


# speedup vs baseline: 58.3483x; 1.0383x over previous
import functools

import jax
import jax.numpy as jnp
from jax.experimental import pallas as pl
from jax.experimental.pallas import tpu as pltpu

K = 20
_INF = 3.0e38


def _body(xq_ref, xall_ref, pts_ref, W1_ref, W2_ref, b1_ref, b2_ref,
          out_ref, D_ref, P_ref, *, blk_q, n):
    q = pl.program_id(1)
    xq = xq_ref[0]
    xall = xall_ref[0]
    sq_all = jnp.sum(xall * xall, axis=0, keepdims=True)
    sq_q = jnp.sum(xq * xq, axis=1, keepdims=True)
    cross = jax.lax.dot_general(
        xq, xall, (((1,), (0,)), ((), ())),
        preferred_element_type=jnp.float32)
    D = sq_q + sq_all - 2.0 * cross
    D_ref[...] = D

    nlev = 5
    nch = n // 128
    sub = 64

    def pyr_body(qq, _):
        rows = pl.ds(qq * sub, sub)
        lev = [jnp.full((sub, 128), _INF, jnp.float32) for _ in range(nlev)]
        for c in range(nch):
            y = D_ref[rows, pl.ds(c * 128, 128)]
            for j in range(nlev):
                lo = jnp.minimum(lev[j], y)
                y = jnp.maximum(lev[j], y)
                lev[j] = lo
        for j in range(nlev):
            P_ref[rows, pl.ds(j * 128, 128)] = lev[j]
        return 0

    jax.lax.fori_loop(0, blk_q // sub, pyr_body, 0, unroll=True)

    def step_small(i, m):
        P = P_ref[...]
        return jnp.min(jnp.where(P > m, P, _INF), axis=1, keepdims=True)

    t = jax.lax.fori_loop(
        0, K + 1, step_small, jnp.full((blk_q, 1), -_INF, jnp.float32))

    deepest = P_ref[:, pl.ds((nlev - 1) * 128, 128)]
    overflow = jnp.max(jnp.where(deepest < t, 1.0, 0.0))

    def slow_t():
        def step_full(i, m):
            Dm = D_ref[...]
            return jnp.min(jnp.where(Dm > m, Dm, _INF), axis=1, keepdims=True)
        return jax.lax.fori_loop(
            0, K + 1, step_full, jnp.full((blk_q, 1), -_INF, jnp.float32))

    t = jax.lax.cond(overflow > 0, slow_t, lambda: t)

    A = (D_ref[...] <= t).astype(jnp.float32)
    f_all = pts_ref[0]
    f_all = jnp.where(f_all > 0, f_all, 0.01 * f_all)
    agg = jax.lax.dot_general(
        A, f_all, (((1,), (0,)), ((), ())),
        preferred_element_type=jnp.float32)
    raw_q = pts_ref[0, pl.ds(q * blk_q, blk_q), :]
    f_q = jnp.where(raw_q > 0, raw_q, 0.01 * raw_q)
    agg_n = agg - f_q
    p1 = jax.lax.dot_general(
        f_q, W1_ref[...], (((1,), (1,)), ((), ())),
        preferred_element_type=jnp.float32)
    p2 = jax.lax.dot_general(
        agg_n, W2_ref[...], (((1,), (1,)), ((), ())),
        preferred_element_type=jnp.float32)
    bias = b1_ref[...] + jnp.float32(K) * b2_ref[...]
    out_ref[0] = (p1 + p2 + bias) * jnp.float32(1.0 / (K + 1)) + raw_q


def kernel(xyz, points, W1, b1, W2, b2):
    B, C, N = points.shape
    blk_q = 2048
    xyzt = jnp.transpose(xyz, (0, 2, 1))
    pts_t = jnp.transpose(points, (0, 2, 1))
    grid = (B, N // blk_q)
    out_t = pl.pallas_call(
        functools.partial(_body, blk_q=blk_q, n=N),
        grid=grid,
        in_specs=[
            pl.BlockSpec((1, blk_q, 3), lambda b, q: (b, q, 0)),
            pl.BlockSpec((1, 3, N), lambda b, q: (b, 0, 0)),
            pl.BlockSpec((1, N, C), lambda b, q: (b, 0, 0)),
            pl.BlockSpec((C, C), lambda b, q: (0, 0)),
            pl.BlockSpec((C, C), lambda b, q: (0, 0)),
            pl.BlockSpec((1, C), lambda b, q: (0, 0)),
            pl.BlockSpec((1, C), lambda b, q: (0, 0)),
        ],
        out_specs=pl.BlockSpec((1, blk_q, C), lambda b, q: (b, q, 0)),
        out_shape=jax.ShapeDtypeStruct((B, N, C), jnp.float32),
        scratch_shapes=[
            pltpu.VMEM((blk_q, N), jnp.float32),
            pltpu.VMEM((blk_q, 5 * 128), jnp.float32),
        ],
    )(xyzt, xyz, pts_t, W1, W2, b1.reshape(1, C), b2.reshape(1, C))
    return jnp.transpose(out_t, (0, 2, 1))
